# baseline (speedup 1.0000x reference)
import jax
import jax.numpy as jnp
from jax import lax
from jax.experimental import pallas as pl
from jax.experimental.pallas import tpu as pltpu
from jax.experimental.pallas import tpu_sc as plsc

NWORDS = 1000000
NTAGS = 1000
EMB = 64
HID = 512
L = 16384
NTILE = 32
T_RANGE = 32768
DUMP = T_RANGE
LAST_FULL = NWORDS // T_RANGE
LAST_LEN = NWORDS - LAST_FULL * T_RANGE

BC = 32768
KSTEPS = (NWORDS + BC - 1) // BC


SEG = T_RANGE + 16


def _sc_body(words_hbm, zeros_hbm, m_hbm, idx_v, lidx_v, ones_v, m_shared,
             sem, zsem):
    cid = lax.axis_index("c")
    sid = lax.axis_index("s")
    t = sid * 2 + cid
    base = t * T_RANGE
    gbase = sid * SEG

    stage = pltpu.async_copy(words_hbm, idx_v, sem)
    zero = pltpu.async_copy(zeros_hbm, m_shared.at[pl.ds(gbase, SEG)], zsem)

    ov = jnp.ones((16,), jnp.float32)

    @pl.loop(0, L // 128)
    def _ones(r):
        for q in range(8):
            ones_v[pl.ds(r * 128 + q * 16, 16)] = ov

    stage.wait()
    zero.wait()

    @pl.loop(0, L // 128)
    def _lidx(r):
        for q in range(8):
            v = idx_v[pl.ds(r * 128 + q * 16, 16)]
            lv = v - base
            ok = (lv >= 0) & (lv < T_RANGE)
            lidx_v[pl.ds(r * 128 + q * 16, 16)] = gbase + jnp.where(
                ok, lv, DUMP)

    pltpu.sync_copy(ones_v, m_shared.at[lidx_v], add=True)

    @pl.when(t < LAST_FULL)
    def _():
        pltpu.sync_copy(m_shared.at[pl.ds(gbase, T_RANGE)],
                        m_hbm.at[pl.ds(base, T_RANGE)])

    @pl.when(t == LAST_FULL)
    def _():
        pltpu.sync_copy(m_shared.at[pl.ds(gbase, LAST_LEN)],
                        m_hbm.at[pl.ds(base, LAST_LEN)])


def _make_sc_counts():
    mesh = plsc.VectorSubcoreMesh(core_axis_name="c", subcore_axis_name="s")
    return pl.kernel(
        _sc_body,
        out_type=jax.ShapeDtypeStruct((NWORDS,), jnp.float32),
        mesh=mesh,
        scratch_types=[
            pltpu.VMEM((L,), jnp.int32),
            pltpu.VMEM((L,), jnp.int32),
            pltpu.VMEM((L,), jnp.float32),
            pltpu.VMEM_SHARED((16 * SEG,), jnp.float32),
            pltpu.SemaphoreType.DMA,
            pltpu.SemaphoreType.DMA,
        ],
        compiler_params=pltpu.CompilerParams(use_tc_tiling_on_sc=False),
    )


def _mv_body(emt_ref, m_ref, w0_ref, b0_ref, w1_ref, b1_ref, wout_ref,
             bout_ref, out_ref, acc_ref):
    k = pl.program_id(0)

    @pl.when(k == 0)
    def _():
        acc_ref[...] = jnp.zeros_like(acc_ref)

    @pl.when(k < KSTEPS - 1)
    def _():
        acc_ref[...] += lax.dot_general(
            m_ref[...].reshape(1, BC), emt_ref[...],
            (((1,), (1,)), ((), ())), preferred_element_type=jnp.float32,
            precision=lax.Precision.HIGHEST)

    @pl.when(k == KSTEPS - 1)
    def _():
        cols = k * BC + lax.broadcasted_iota(jnp.int32, (1, BC), 1)
        valid = cols < NWORDS
        mv = jnp.where(valid, m_ref[...].reshape(1, BC), 0.0)
        eb = jnp.where(valid, emt_ref[...], 0.0)
        s = acc_ref[...] + lax.dot_general(
            mv, eb, (((1,), (1,)), ((), ())),
            preferred_element_type=jnp.float32,
            precision=lax.Precision.HIGHEST)
        h = jnp.tanh(
            lax.dot_general(s, w0_ref[...], (((1,), (1,)), ((), ())),
                            preferred_element_type=jnp.float32,
            precision=lax.Precision.HIGHEST) + b0_ref[...])
        h = jnp.tanh(
            lax.dot_general(h, w1_ref[...], (((1,), (1,)), ((), ())),
                            preferred_element_type=jnp.float32,
            precision=lax.Precision.HIGHEST) + b1_ref[...])
        out_ref[...] = lax.dot_general(
            h, wout_ref[...], (((1,), (1,)), ((), ())),
            preferred_element_type=jnp.float32,
            precision=lax.Precision.HIGHEST) + bout_ref[...]


_mv_call = pl.pallas_call(
    _mv_body,
    grid=(KSTEPS,),
    in_specs=[
        pl.BlockSpec((EMB, BC), lambda k: (0, k)),
        pl.BlockSpec((BC,), lambda k: (k,)),
        pl.BlockSpec((HID, EMB), lambda k: (0, 0)),
        pl.BlockSpec((1, HID), lambda k: (0, 0)),
        pl.BlockSpec((HID, HID), lambda k: (0, 0)),
        pl.BlockSpec((1, HID), lambda k: (0, 0)),
        pl.BlockSpec((NTAGS, HID), lambda k: (0, 0)),
        pl.BlockSpec((1, NTAGS), lambda k: (0, 0)),
    ],
    out_specs=pl.BlockSpec((1, NTAGS), lambda k: (0, 0)),
    out_shape=jax.ShapeDtypeStruct((1, NTAGS), jnp.float32),
    scratch_shapes=[pltpu.VMEM((1, EMB), jnp.float32)],
)


@jax.jit
def kernel(words, emb, W0, b0, W1, b1, Wout, bout):
    words1 = words.astype(jnp.int32)
    zeros = jnp.zeros((SEG,), jnp.float32)
    m = _make_sc_counts()(words1, zeros)
    emt = emb.T
    return _mv_call(emt, m, W0, b0.reshape(1, HID), W1, b1.reshape(1, HID),
                    Wout, bout.reshape(1, NTAGS))


# speedup vs baseline: 2.0205x; 2.0205x over previous
import jax
import jax.numpy as jnp
from jax import lax
from jax.experimental import pallas as pl
from jax.experimental.pallas import tpu as pltpu
from jax.experimental.pallas import tpu_sc as plsc

NWORDS = 1000000
NTAGS = 1000
EMB = 64
HID = 512
L = 16384
NTILE = 32
T_RANGE = 32768
DUMP = T_RANGE
LAST_FULL = NWORDS // T_RANGE
LAST_LEN = NWORDS - LAST_FULL * T_RANGE

BC = 32768
KSTEPS = (NWORDS + BC - 1) // BC


SEG = T_RANGE + 16


def _sc_body(words_hbm, zeros_hbm, m_hbm, idx_v, lidx_v, ones_v, m_shared,
             sem, zsem):
    cid = lax.axis_index("c")
    sid = lax.axis_index("s")
    t = sid * 2 + cid
    base = t * T_RANGE
    gbase = sid * SEG

    stage = pltpu.async_copy(words_hbm, idx_v, sem)
    zero = pltpu.async_copy(zeros_hbm, m_shared.at[pl.ds(gbase, SEG)], zsem)

    ov = jnp.ones((16,), jnp.float32)

    @pl.loop(0, L // 128)
    def _ones(r):
        for q in range(8):
            ones_v[pl.ds(r * 128 + q * 16, 16)] = ov

    stage.wait()
    zero.wait()

    @pl.loop(0, L // 128)
    def _lidx(r):
        for q in range(8):
            v = idx_v[pl.ds(r * 128 + q * 16, 16)]
            lv = v - base
            ok = (lv >= 0) & (lv < T_RANGE)
            lidx_v[pl.ds(r * 128 + q * 16, 16)] = gbase + jnp.where(
                ok, lv, DUMP)

    pltpu.sync_copy(ones_v, m_shared.at[lidx_v], add=True)

    @pl.when(t < LAST_FULL)
    def _():
        pltpu.sync_copy(m_shared.at[pl.ds(gbase, T_RANGE)],
                        m_hbm.at[pl.ds(base, T_RANGE)])

    @pl.when(t == LAST_FULL)
    def _():
        pltpu.sync_copy(m_shared.at[pl.ds(gbase, LAST_LEN)],
                        m_hbm.at[pl.ds(base, LAST_LEN)])


def _make_sc_counts():
    mesh = plsc.VectorSubcoreMesh(core_axis_name="c", subcore_axis_name="s")
    return pl.kernel(
        _sc_body,
        out_type=jax.ShapeDtypeStruct((NWORDS,), jnp.float32),
        mesh=mesh,
        scratch_types=[
            pltpu.VMEM((L,), jnp.int32),
            pltpu.VMEM((L,), jnp.int32),
            pltpu.VMEM((L,), jnp.float32),
            pltpu.VMEM_SHARED((16 * SEG,), jnp.float32),
            pltpu.SemaphoreType.DMA,
            pltpu.SemaphoreType.DMA,
        ],
        compiler_params=pltpu.CompilerParams(use_tc_tiling_on_sc=False),
    )


def _mv_body(emt_ref, m_ref, w0_ref, b0_ref, w1_ref, b1_ref, wout_ref,
             bout_ref, out_ref, acc_ref):
    k = pl.program_id(0)

    @pl.when(k == 0)
    def _():
        acc_ref[...] = jnp.zeros_like(acc_ref)

    @pl.when(k < KSTEPS - 1)
    def _():
        acc_ref[...] += lax.dot_general(
            m_ref[...].reshape(1, BC), emt_ref[...],
            (((1,), (1,)), ((), ())), preferred_element_type=jnp.float32)

    @pl.when(k == KSTEPS - 1)
    def _():
        cols = k * BC + lax.broadcasted_iota(jnp.int32, (1, BC), 1)
        valid = cols < NWORDS
        mv = jnp.where(valid, m_ref[...].reshape(1, BC), 0.0)
        eb = jnp.where(valid, emt_ref[...], 0.0)
        s = acc_ref[...] + lax.dot_general(
            mv, eb, (((1,), (1,)), ((), ())),
            preferred_element_type=jnp.float32)
        h = jnp.tanh(
            lax.dot_general(s, w0_ref[...], (((1,), (1,)), ((), ())),
                            preferred_element_type=jnp.float32,
            precision=lax.Precision.HIGHEST) + b0_ref[...])
        h = jnp.tanh(
            lax.dot_general(h, w1_ref[...], (((1,), (1,)), ((), ())),
                            preferred_element_type=jnp.float32,
            precision=lax.Precision.HIGHEST) + b1_ref[...])
        out_ref[...] = lax.dot_general(
            h, wout_ref[...], (((1,), (1,)), ((), ())),
            preferred_element_type=jnp.float32,
            precision=lax.Precision.HIGHEST) + bout_ref[...]


_mv_call = pl.pallas_call(
    _mv_body,
    grid=(KSTEPS,),
    in_specs=[
        pl.BlockSpec((EMB, BC), lambda k: (0, k)),
        pl.BlockSpec((BC,), lambda k: (k,)),
        pl.BlockSpec((HID, EMB), lambda k: (0, 0)),
        pl.BlockSpec((1, HID), lambda k: (0, 0)),
        pl.BlockSpec((HID, HID), lambda k: (0, 0)),
        pl.BlockSpec((1, HID), lambda k: (0, 0)),
        pl.BlockSpec((NTAGS, HID), lambda k: (0, 0)),
        pl.BlockSpec((1, NTAGS), lambda k: (0, 0)),
    ],
    out_specs=pl.BlockSpec((1, NTAGS), lambda k: (0, 0)),
    out_shape=jax.ShapeDtypeStruct((1, NTAGS), jnp.float32),
    scratch_shapes=[pltpu.VMEM((1, EMB), jnp.float32)],
)


@jax.jit
def kernel(words, emb, W0, b0, W1, b1, Wout, bout):
    words1 = words.astype(jnp.int32)
    zeros = jnp.zeros((SEG,), jnp.float32)
    m = _make_sc_counts()(words1, zeros)
    emt = emb.T
    return _mv_call(emt, m, W0, b0.reshape(1, HID), W1, b1.reshape(1, HID),
                    Wout, bout.reshape(1, NTAGS))


# speedup vs baseline: 2.2364x; 1.1068x over previous
import jax
import jax.numpy as jnp
from jax import lax
from jax.experimental import pallas as pl
from jax.experimental.pallas import tpu as pltpu
from jax.experimental.pallas import tpu_sc as plsc

NWORDS = 1000000
NTAGS = 1000
EMB = 64
HID = 512
L = 16384
NW = 32
B_PER_W = L // NW

BC = 32768
KSTEPS = (NWORDS + BC - 1) // BC
REGION = KSTEPS * BC
STRIPE = 65536
LAST_SID = 15
LAST_LEN = NWORDS - LAST_SID * STRIPE


def _sc_body(words_hbm, zeros_hbm, m_hbm, idx_v, ones_v, m_shared, sem):
    cid = lax.axis_index("c")
    sid = lax.axis_index("s")
    w = sid * 2 + cid

    stage = pltpu.async_copy(words_hbm.at[pl.ds(w * B_PER_W, B_PER_W)],
                             idx_v, sem)

    ov = jnp.ones((16,), jnp.float32)

    @pl.loop(0, B_PER_W // 16, unroll=8)
    def _ones(r):
        ones_v[pl.ds(r * 16, 16)] = ov

    @pl.when(sid < LAST_SID)
    def _():
        pltpu.sync_copy(zeros_hbm, m_shared.at[pl.ds(sid * STRIPE, STRIPE)])

    @pl.when(sid == LAST_SID)
    def _():
        pltpu.sync_copy(zeros_hbm.at[pl.ds(0, LAST_LEN)],
                        m_shared.at[pl.ds(LAST_SID * STRIPE, LAST_LEN)])

    stage.wait()
    plsc.subcore_barrier()

    pltpu.sync_copy(ones_v, m_shared.at[idx_v], add=True)
    plsc.subcore_barrier()

    out0 = cid * REGION

    @pl.when(sid < LAST_SID)
    def _():
        pltpu.sync_copy(m_shared.at[pl.ds(sid * STRIPE, STRIPE)],
                        m_hbm.at[pl.ds(out0 + sid * STRIPE, STRIPE)])

    @pl.when(sid == LAST_SID)
    def _():
        pltpu.sync_copy(m_shared.at[pl.ds(LAST_SID * STRIPE, LAST_LEN)],
                        m_hbm.at[pl.ds(out0 + LAST_SID * STRIPE, LAST_LEN)])


def _make_sc_counts():
    mesh = plsc.VectorSubcoreMesh(core_axis_name="c", subcore_axis_name="s")
    return pl.kernel(
        _sc_body,
        out_type=jax.ShapeDtypeStruct((2 * REGION,), jnp.float32),
        mesh=mesh,
        scratch_types=[
            pltpu.VMEM((B_PER_W,), jnp.int32),
            pltpu.VMEM((B_PER_W,), jnp.float32),
            pltpu.VMEM_SHARED((NWORDS,), jnp.float32),
            pltpu.SemaphoreType.DMA,
        ],
        compiler_params=pltpu.CompilerParams(use_tc_tiling_on_sc=False),
    )


def _mv_body(emt_ref, m0_ref, m1_ref, w0_ref, b0_ref, w1_ref, b1_ref,
             wout_ref, bout_ref, out_ref, acc_ref):
    k = pl.program_id(0)

    @pl.when(k == 0)
    def _():
        acc_ref[...] = jnp.zeros_like(acc_ref)

    @pl.when(k < KSTEPS - 1)
    def _():
        mv = (m0_ref[...] + m1_ref[...]).reshape(1, BC)
        acc_ref[...] += lax.dot_general(
            mv, emt_ref[...],
            (((1,), (1,)), ((), ())), preferred_element_type=jnp.float32)

    @pl.when(k == KSTEPS - 1)
    def _():
        cols = k * BC + lax.broadcasted_iota(jnp.int32, (1, BC), 1)
        valid = cols < NWORDS
        mv = jnp.where(valid, (m0_ref[...] + m1_ref[...]).reshape(1, BC), 0.0)
        eb = jnp.where(valid, emt_ref[...], 0.0)
        s = acc_ref[...] + lax.dot_general(
            mv, eb, (((1,), (1,)), ((), ())),
            preferred_element_type=jnp.float32)
        h = jnp.tanh(
            lax.dot_general(s, w0_ref[...], (((1,), (1,)), ((), ())),
                            preferred_element_type=jnp.float32,
                            precision=lax.Precision.HIGHEST) + b0_ref[...])
        h = jnp.tanh(
            lax.dot_general(h, w1_ref[...], (((1,), (1,)), ((), ())),
                            preferred_element_type=jnp.float32,
                            precision=lax.Precision.HIGHEST) + b1_ref[...])
        out_ref[...] = lax.dot_general(
            h, wout_ref[...], (((1,), (1,)), ((), ())),
            preferred_element_type=jnp.float32,
            precision=lax.Precision.HIGHEST) + bout_ref[...]


_mv_call = pl.pallas_call(
    _mv_body,
    grid=(KSTEPS,),
    in_specs=[
        pl.BlockSpec((EMB, BC), lambda k: (0, k)),
        pl.BlockSpec((BC,), lambda k: (k,)),
        pl.BlockSpec((BC,), lambda k: (KSTEPS + k,)),
        pl.BlockSpec((HID, EMB), lambda k: (0, 0)),
        pl.BlockSpec((1, HID), lambda k: (0, 0)),
        pl.BlockSpec((HID, HID), lambda k: (0, 0)),
        pl.BlockSpec((1, HID), lambda k: (0, 0)),
        pl.BlockSpec((NTAGS, HID), lambda k: (0, 0)),
        pl.BlockSpec((1, NTAGS), lambda k: (0, 0)),
    ],
    out_specs=pl.BlockSpec((1, NTAGS), lambda k: (0, 0)),
    out_shape=jax.ShapeDtypeStruct((1, NTAGS), jnp.float32),
    scratch_shapes=[pltpu.VMEM((1, EMB), jnp.float32)],
)


@jax.jit
def kernel(words, emb, W0, b0, W1, b1, Wout, bout):
    words1 = words.astype(jnp.int32)
    zeros = jnp.zeros((STRIPE,), jnp.float32)
    m = _make_sc_counts()(words1, zeros)
    emt = emb.T
    return _mv_call(emt, m, m, W0, b0.reshape(1, HID), W1,
                    b1.reshape(1, HID), Wout, bout.reshape(1, NTAGS))
